# baseline (speedup 1.0000x reference)
import functools

import jax
import jax.numpy as jnp
import numpy as np
from jax.experimental import pallas as pl


def _pass1_body(n, r_ref, z_ref, v_ref, m_ref, w1t_ref, w1_ref, embtbf_ref,
                w2c_ref, b1c_ref, st_ref, ep_ref, kin_ref, pv_ref):
    i = pl.program_id(0)

    @pl.when(i == 0)
    def _init():
        ep_ref[...] = jnp.zeros_like(ep_ref)
        kin_ref[...] = jnp.zeros_like(kin_ref)
        pv_ref[...] = jnp.zeros_like(pv_ref)

    bbk = z_ref.shape[0]
    lane = jax.lax.broadcasted_iota(jnp.int32, (1, bbk), 1)
    mask = (i * bbk + lane) < n

    rt = r_ref[...].T
    x = jnp.dot(w1t_ref[...], rt, preferred_element_type=jnp.float32)

    z = z_ref[...].reshape(1, bbk)
    nsp = embtbf_ref.shape[1]
    iota_s = jax.lax.broadcasted_iota(jnp.int32, (nsp, bbk), 0)
    oh = (iota_s == z).astype(jnp.bfloat16)
    embz = jnp.dot(embtbf_ref[...], oh, preferred_element_type=jnp.float32)

    h = jnp.tanh(x + b1c_ref[...] + embz)
    w2c = w2c_ref[...]
    ep_ref[...] += jnp.sum(jnp.where(mask, h * w2c, 0.0)).reshape(1, 1)

    u = (1.0 - h * h) * w2c
    eg = jnp.dot(w1_ref[...], u, preferred_element_type=jnp.float32)

    vt = v_ref[...].T
    mrow = m_ref[...].reshape(1, bbk)
    mv = vt * mrow
    st_ref[...] = jnp.concatenate([mv, eg], axis=0)
    kin_ref[...] += jnp.sum(jnp.where(mask, mv * vt, 0.0)).reshape(1, 1)
    pv_ref[...] += jnp.sum(jnp.where(mask, mv, 0.0), axis=1,
                           keepdims=True)


def _pass2_body(st_ref, m_ref, es3_ref, ps3_ref, j_ref):
    es3 = es3_ref[...]
    ps3 = ps3_ref[...]
    mrow = m_ref[...].reshape(1, m_ref.shape[0])
    st = st_ref[...]
    jvt = st[0:3, :] * es3 + mrow * ps3
    jrt = st[3:6, :] * es3
    jt = jnp.concatenate([jvt, jrt], axis=0)
    j_ref[...] = jt.T


def _cdiv(a, b):
    return (a + b - 1) // b


@jax.jit
def kernel(r, v, batch, z, m, E0, W1, b1, emb, W2, b2):
    n = r.shape[0]
    bb = 8192
    grid = _cdiv(n, bb)

    w1t = W1.T
    embtbf = emb.T.astype(jnp.bfloat16)
    b1c = b1[:, None]
    w2c = W2

    row6 = pl.BlockSpec((6, bb), lambda i: (0, i))
    full = lambda a: pl.BlockSpec(a.shape, lambda i: (0, 0))

    st, ep, kin, pv = pl.pallas_call(
        functools.partial(_pass1_body, n),
        grid=(grid,),
        in_specs=[
            pl.BlockSpec((bb, 3), lambda i: (i, 0)),
            pl.BlockSpec((bb,), lambda i: (i,)),
            pl.BlockSpec((bb, 3), lambda i: (i, 0)),
            pl.BlockSpec((bb,), lambda i: (i,)),
            full(w1t), full(W1), full(embtbf), full(w2c), full(b1c),
        ],
        out_specs=[
            row6,
            pl.BlockSpec((1, 1), lambda i: (0, 0)),
            pl.BlockSpec((1, 1), lambda i: (0, 0)),
            pl.BlockSpec((3, 1), lambda i: (0, 0)),
        ],
        out_shape=[
            jax.ShapeDtypeStruct((6, n), jnp.float32),
            jax.ShapeDtypeStruct((1, 1), jnp.float32),
            jax.ShapeDtypeStruct((1, 1), jnp.float32),
            jax.ShapeDtypeStruct((3, 1), jnp.float32),
        ],
    )(r, z, v, m, w1t, W1, embtbf, w2c, b1c)

    e_pot = ep[0, 0] + n * b2[0]
    e_kin = 0.5 * kin[0, 0]
    e_val = e_pot + e_kin - E0[0, 0]
    c = jnp.concatenate([e_val.reshape(1, 1), pv], axis=0)

    es3 = jnp.broadcast_to(e_val.reshape(1, 1), (3, 1))

    j = pl.pallas_call(
        _pass2_body,
        grid=(grid,),
        in_specs=[
            row6,
            pl.BlockSpec((bb,), lambda i: (i,)),
            full(es3), full(pv),
        ],
        out_specs=pl.BlockSpec((bb, 6), lambda i: (i, 0)),
        out_shape=jax.ShapeDtypeStruct((n, 6), jnp.float32),
    )(st, m, es3, pv)

    return (c, j)


# speedup vs baseline: 1.0629x; 1.0629x over previous
import functools

import jax
import jax.numpy as jnp
import numpy as np
from jax.experimental import pallas as pl


def _pass1_body(n, r_ref, z_ref, v_ref, m_ref, w1t_ref, w1_ref, embtbf_ref,
                w2c_ref, b1c_ref, st_ref, ep_ref, kin_ref, pv_ref):
    i = pl.program_id(0)

    @pl.when(i == 0)
    def _init():
        ep_ref[...] = jnp.zeros_like(ep_ref)
        kin_ref[...] = jnp.zeros_like(kin_ref)
        pv_ref[...] = jnp.zeros_like(pv_ref)

    bbk = z_ref.shape[0]
    lane = jax.lax.broadcasted_iota(jnp.int32, (1, bbk), 1)
    mask = (i * bbk + lane) < n

    rt = r_ref[...].T
    x = jnp.dot(w1t_ref[...], rt, preferred_element_type=jnp.float32)

    z = z_ref[...].reshape(1, bbk)
    nsp = embtbf_ref.shape[1]
    iota_s = jax.lax.broadcasted_iota(jnp.int32, (nsp, bbk), 0)
    oh = (iota_s == z).astype(jnp.bfloat16)
    embz = jnp.dot(embtbf_ref[...], oh, preferred_element_type=jnp.float32)

    h = jnp.tanh(x + b1c_ref[...] + embz)
    w2c = w2c_ref[...]
    ep_ref[...] += jnp.sum(jnp.where(mask, h * w2c, 0.0)).reshape(1, 1)

    u = (1.0 - h * h) * w2c
    eg = jnp.dot(w1_ref[...], u, preferred_element_type=jnp.float32)

    vt = v_ref[...].T
    mrow = m_ref[...].reshape(1, bbk)
    mv = vt * mrow
    st_ref[...] = jnp.concatenate([mv, eg], axis=0)
    kin_ref[...] += jnp.sum(jnp.where(mask, mv * vt, 0.0)).reshape(1, 1)
    pv_ref[...] += jnp.sum(jnp.where(mask, mv, 0.0), axis=1,
                           keepdims=True)


def _pass2_body(st_ref, m_ref, es3_ref, ps3_ref, j_ref):
    es3 = es3_ref[...]
    ps3 = ps3_ref[...]
    mrow = m_ref[...].reshape(1, m_ref.shape[0])
    st = st_ref[...]
    jvt = st[0:3, :] * es3 + mrow * ps3
    jrt = st[3:6, :] * es3
    jt = jnp.concatenate([jvt, jrt], axis=0)
    j_ref[...] = jt.T


def _cdiv(a, b):
    return (a + b - 1) // b


@jax.jit
def kernel(r, v, batch, z, m, E0, W1, b1, emb, W2, b2):
    n = r.shape[0]
    bb = 16384
    grid = _cdiv(n, bb)

    w1t = W1.T
    embtbf = emb.T.astype(jnp.bfloat16)
    b1c = b1[:, None]
    w2c = W2

    row6 = pl.BlockSpec((6, bb), lambda i: (0, i))
    full = lambda a: pl.BlockSpec(a.shape, lambda i: (0, 0))

    st, ep, kin, pv = pl.pallas_call(
        functools.partial(_pass1_body, n),
        grid=(grid,),
        in_specs=[
            pl.BlockSpec((bb, 3), lambda i: (i, 0)),
            pl.BlockSpec((bb,), lambda i: (i,)),
            pl.BlockSpec((bb, 3), lambda i: (i, 0)),
            pl.BlockSpec((bb,), lambda i: (i,)),
            full(w1t), full(W1), full(embtbf), full(w2c), full(b1c),
        ],
        out_specs=[
            row6,
            pl.BlockSpec((1, 1), lambda i: (0, 0)),
            pl.BlockSpec((1, 1), lambda i: (0, 0)),
            pl.BlockSpec((3, 1), lambda i: (0, 0)),
        ],
        out_shape=[
            jax.ShapeDtypeStruct((6, n), jnp.float32),
            jax.ShapeDtypeStruct((1, 1), jnp.float32),
            jax.ShapeDtypeStruct((1, 1), jnp.float32),
            jax.ShapeDtypeStruct((3, 1), jnp.float32),
        ],
    )(r, z, v, m, w1t, W1, embtbf, w2c, b1c)

    e_pot = ep[0, 0] + n * b2[0]
    e_kin = 0.5 * kin[0, 0]
    e_val = e_pot + e_kin - E0[0, 0]
    c = jnp.concatenate([e_val.reshape(1, 1), pv], axis=0)

    es3 = jnp.broadcast_to(e_val.reshape(1, 1), (3, 1))

    j = pl.pallas_call(
        _pass2_body,
        grid=(grid,),
        in_specs=[
            row6,
            pl.BlockSpec((bb,), lambda i: (i,)),
            full(es3), full(pv),
        ],
        out_specs=pl.BlockSpec((bb, 6), lambda i: (i, 0)),
        out_shape=jax.ShapeDtypeStruct((n, 6), jnp.float32),
    )(st, m, es3, pv)

    return (c, j)
